# Initial kernel scaffold; baseline (speedup 1.0000x reference)
import jax
import jax.numpy as jnp
from jax.experimental import pallas as pl


def kernel(object_coords, states_objects, mask_edge, class_table, W_state, b_state, W_c1, b_c1, W_c2, b_c2, W_comb, b_comb, W_eb, b_eb, W_pp, b_pp, class_objects, from_indices_onehot, to_indices_onehot):
    raise NotImplementedError("write your pallas kernel here")


# speedup vs baseline: 1.2796x; 1.2796x over previous
import functools

import jax
import jax.numpy as jnp
from jax.experimental import pallas as pl
from jax.experimental.pallas import tpu as pltpu

_B, _T, _N, _H = 16, 16, 150, 256
_H2 = _H // 2
_ROWS = _B * _T * _N
_BLK = 2560
_CPAD = 128


def _body(oc_ref, st_ref, ids_ref, ct_ref,
          wc1_ref, bc1_ref, wc2_ref, bc2_ref,
          wst_ref, bst_ref,
          wcb_class_ref, wcb_coord_ref, wcb_state_ref, bcb_ref,
          wpp_node_ref, wpp_x_ref, bpp_ref,
          out_ref, tc_ref):
    f32 = jnp.float32

    @pl.when(pl.program_id(0) == 0)
    def _():
        tc_ref[...] = jnp.dot(jnp.maximum(ct_ref[...], 0.0), wcb_class_ref[...],
                              preferred_element_type=f32)

    h1 = jnp.maximum(jnp.dot(oc_ref[...], wc1_ref[...],
                             preferred_element_type=f32) + bc1_ref[...], 0.0)
    coord_emb = jnp.dot(h1, wc2_ref[...], preferred_element_type=f32) + bc2_ref[...]
    state_emb = jnp.dot(st_ref[...], wst_ref[...],
                        preferred_element_type=f32) + bst_ref[...]

    onehot = (ids_ref[...] ==
              jax.lax.broadcasted_iota(jnp.int32, (_BLK, _CPAD), 1)).astype(f32)
    x = (jnp.dot(onehot, tc_ref[...], preferred_element_type=f32)
         + jnp.dot(jnp.maximum(coord_emb, 0.0), wcb_coord_ref[...],
                   preferred_element_type=f32)
         + jnp.dot(jnp.maximum(state_emb, 0.0), wcb_state_ref[...],
                   preferred_element_type=f32)
         + bcb_ref[...])

    y = jnp.dot(x, wpp_x_ref[...], preferred_element_type=f32) + bpp_ref[...]
    e = jnp.maximum(y, 0.0)
    e = jnp.maximum(jnp.dot(e, wpp_node_ref[...], preferred_element_type=f32) + y, 0.0)
    e = jnp.maximum(jnp.dot(e, wpp_node_ref[...], preferred_element_type=f32) + y, 0.0)
    out_ref[...] = e


@functools.partial(jax.jit, static_argnames=())
def kernel(object_coords, states_objects, mask_edge, class_table,
           W_state, b_state, W_c1, b_c1, W_c2, b_c2, W_comb, b_comb,
           W_eb, b_eb, W_pp, b_pp,
           class_objects, from_indices_onehot, to_indices_onehot):
    del mask_edge, W_eb, b_eb, from_indices_onehot, to_indices_onehot

    oc = jnp.pad(object_coords.reshape(_ROWS, 6), ((0, 0), (0, 2)))
    st = jnp.pad(states_objects.reshape(_ROWS, 4), ((0, 0), (0, 4)))
    ids = class_objects.reshape(_ROWS, 1)
    ct = jnp.pad(class_table, ((0, _CPAD - class_table.shape[0]), (0, 0)))
    wc1 = jnp.pad(W_c1, ((0, 2), (0, 0)))
    wst = jnp.pad(W_state, ((0, 4), (0, 0)))

    row2 = lambda v: v.reshape(1, -1)
    wfull = lambda s: pl.BlockSpec(s, lambda i: (0, 0))

    grid = (_ROWS // _BLK,)
    out = pl.pallas_call(
        _body,
        grid=grid,
        in_specs=[
            pl.BlockSpec((_BLK, 8), lambda i: (i, 0)),
            pl.BlockSpec((_BLK, 8), lambda i: (i, 0)),
            pl.BlockSpec((_BLK, 1), lambda i: (i, 0)),
            wfull((_CPAD, _H2)),
            wfull((8, _H2)), wfull((1, _H2)),
            wfull((_H2, _H2)), wfull((1, _H2)),
            wfull((8, _H2)), wfull((1, _H2)),
            wfull((_H2, _H)),
            wfull((_H2, _H)),
            wfull((_H2, _H)),
            wfull((1, _H)),
            wfull((_H, _H)),
            wfull((_H, _H)),
            wfull((1, _H)),
        ],
        out_specs=pl.BlockSpec((_BLK, _H), lambda i: (i, 0)),
        out_shape=jax.ShapeDtypeStruct((_ROWS, _H), jnp.float32),
        scratch_shapes=[pltpu.VMEM((_CPAD, _H), jnp.float32)],
    )(oc, st, ids, ct,
      wc1, row2(b_c1), W_c2, row2(b_c2),
      wst, row2(b_state),
      W_comb[:_H2], W_comb[_H2:2 * _H2], W_comb[2 * _H2:], row2(b_comb),
      W_pp[:_H], W_pp[_H:], row2(b_pp))
    return out.reshape(_B, _T, _N, _H)


# speedup vs baseline: 1.3709x; 1.0713x over previous
import functools

import jax
import jax.numpy as jnp
from jax.experimental import pallas as pl
from jax.experimental.pallas import tpu as pltpu

_B, _T, _N, _H = 16, 16, 150, 256
_H2 = _H // 2
_ROWS = _B * _T * _N
_BLK = 2560
_CPAD = 128


def _body(oc_ref, st_ref, ids_ref, ct_ref,
          wc1_ref, bc1_ref, wc2_ref, bc2_ref,
          wst_ref, bst_ref,
          wcb_class_ref, wcb_coord_ref, wcb_state_ref, bcb_ref,
          wpp_node_ref, wpp_x_ref, bpp_ref,
          out_ref, tc_ref):
    f32 = jnp.float32
    bf16 = jnp.bfloat16

    def mm(a, b_ref):
        return jnp.dot(a.astype(bf16), b_ref[...], preferred_element_type=f32)

    @pl.when(pl.program_id(0) == 0)
    def _():
        tc_ref[...] = mm(jnp.maximum(ct_ref[...], 0.0), wcb_class_ref).astype(bf16)

    h1 = jnp.maximum(mm(oc_ref[...], wc1_ref) + bc1_ref[...], 0.0)
    coord_emb = mm(h1, wc2_ref) + bc2_ref[...]
    state_emb = mm(st_ref[...], wst_ref) + bst_ref[...]

    onehot = (ids_ref[...] ==
              jax.lax.broadcasted_iota(jnp.int32, (_BLK, _CPAD), 1)).astype(bf16)
    x = (jnp.dot(onehot, tc_ref[...], preferred_element_type=f32)
         + mm(jnp.maximum(coord_emb, 0.0), wcb_coord_ref)
         + mm(jnp.maximum(state_emb, 0.0), wcb_state_ref)
         + bcb_ref[...])

    y = mm(x, wpp_x_ref) + bpp_ref[...]
    e = jnp.maximum(y, 0.0)
    e = jnp.maximum(mm(e, wpp_node_ref) + y, 0.0)
    e = jnp.maximum(mm(e, wpp_node_ref) + y, 0.0)
    out_ref[...] = e


@functools.partial(jax.jit, static_argnames=())
def kernel(object_coords, states_objects, mask_edge, class_table,
           W_state, b_state, W_c1, b_c1, W_c2, b_c2, W_comb, b_comb,
           W_eb, b_eb, W_pp, b_pp,
           class_objects, from_indices_onehot, to_indices_onehot):
    del mask_edge, W_eb, b_eb, from_indices_onehot, to_indices_onehot

    bf16 = jnp.bfloat16
    oc = jnp.pad(object_coords.reshape(_ROWS, 6), ((0, 0), (0, 2))).astype(bf16)
    st = jnp.pad(states_objects.reshape(_ROWS, 4), ((0, 0), (0, 4))).astype(bf16)
    ids = class_objects.reshape(_ROWS, 1)
    ct = jnp.pad(class_table, ((0, _CPAD - class_table.shape[0]), (0, 0)))
    wc1 = jnp.pad(W_c1, ((0, 2), (0, 0))).astype(bf16)
    wst = jnp.pad(W_state, ((0, 4), (0, 0))).astype(bf16)

    row2 = lambda v: v.reshape(1, -1)
    wfull = lambda s: pl.BlockSpec(s, lambda i: (0, 0))

    grid = (_ROWS // _BLK,)
    out = pl.pallas_call(
        _body,
        grid=grid,
        in_specs=[
            pl.BlockSpec((_BLK, 8), lambda i: (i, 0)),
            pl.BlockSpec((_BLK, 8), lambda i: (i, 0)),
            pl.BlockSpec((_BLK, 1), lambda i: (i, 0)),
            wfull((_CPAD, _H2)),
            wfull((8, _H2)), wfull((1, _H2)),
            wfull((_H2, _H2)), wfull((1, _H2)),
            wfull((8, _H2)), wfull((1, _H2)),
            wfull((_H2, _H)),
            wfull((_H2, _H)),
            wfull((_H2, _H)),
            wfull((1, _H)),
            wfull((_H, _H)),
            wfull((_H, _H)),
            wfull((1, _H)),
        ],
        out_specs=pl.BlockSpec((_BLK, _H), lambda i: (i, 0)),
        out_shape=jax.ShapeDtypeStruct((_ROWS, _H), jnp.float32),
        scratch_shapes=[pltpu.VMEM((_CPAD, _H), bf16)],
    )(oc, st, ids, ct,
      wc1, row2(b_c1), W_c2.astype(bf16), row2(b_c2),
      wst, row2(b_state),
      W_comb[:_H2].astype(bf16), W_comb[_H2:2 * _H2].astype(bf16),
      W_comb[2 * _H2:].astype(bf16), row2(b_comb),
      W_pp[:_H].astype(bf16), W_pp[_H:].astype(bf16), row2(b_pp))
    return out.reshape(_B, _T, _N, _H)
